# baseline (speedup 1.0000x reference)
import functools

import jax
import jax.numpy as jnp
from jax import lax
from jax.experimental import pallas as pl
from jax.experimental.pallas import tpu as pltpu
from jax.experimental.pallas import tpu_sc as plsc

N = 10000
D = 128
E = 320000
NUM_LAYERS = 3
BN_EPS = 1e-5

NC = 2
NS = 16
NW = NC * NS
NP = 10240
EPW = 10240
EPAD = NW * EPW
C = 80
NCH = EPW // C
RPT = NP // NS


@functools.cache
def _build_sc_kernels():
    mesh = plsc.VectorSubcoreMesh(core_axis_name="c", subcore_axis_name="s")

    @functools.partial(
        pl.kernel,
        out_type=jax.ShapeDtypeStruct((NC, NP, D), jnp.float32),
        mesh=mesh,
        scratch_types=[
            pltpu.VMEM((EPW,), jnp.int32),
            pltpu.VMEM((NCH, C), jnp.int32),
            pltpu.VMEM((C, D), jnp.float32),
            pltpu.VMEM((C, D), jnp.float32),
            pltpu.VMEM_SHARED((NP, D), jnp.float32),
            pltpu.SemaphoreType.DMA,
            pltpu.SemaphoreType.DMA,
        ],
    )
    def sc_scatter(m_hbm, row_hbm, col_hbm, zeros_hbm, out_hbm,
                   ridx, cidx, g0, g1, acc, sem0, sem1):
        cid = lax.axis_index("c")
        sid = lax.axis_index("s")
        wid = sid * NC + cid
        pltpu.sync_copy(zeros_hbm, acc.at[pl.ds(sid * RPT, RPT)])
        pltpu.sync_copy(row_hbm.at[wid], ridx)
        pltpu.sync_copy(col_hbm.at[wid], cidx)
        plsc.subcore_barrier()

        H = C // 2

        def fire(j, g, sem):
            pltpu.async_copy(m_hbm.at[ridx.at[pl.ds(j * C, H)]],
                             g.at[pl.ds(0, H)], sem)
            pltpu.async_copy(m_hbm.at[ridx.at[pl.ds(j * C + H, H)]],
                             g.at[pl.ds(H, H)], sem)

        def drain(g, sem):
            pltpu.make_async_copy(m_hbm.at[ridx.at[pl.ds(0, C)]], g, sem).wait()

        fire(0, g0, sem0)

        def body(i, carry):
            ja = 2 * i
            jb = 2 * i + 1
            jn = jnp.minimum(jb + 1, NCH - 1)
            drain(g0, sem0)
            fire(jb, g1, sem1)
            pltpu.sync_copy(g0, acc.at[cidx.at[ja]], add=True)
            drain(g1, sem1)
            fire(jn, g0, sem0)
            pltpu.sync_copy(g1, acc.at[cidx.at[jb]], add=True)
            return carry

        lax.fori_loop(0, NCH // 2, body, 0)
        drain(g0, sem0)
        plsc.subcore_barrier()
        pltpu.sync_copy(acc.at[pl.ds(sid * RPT, RPT)],
                        out_hbm.at[cid, pl.ds(sid * RPT, RPT)])

    @functools.partial(
        pl.kernel,
        out_type=jax.ShapeDtypeStruct((NC, NP), jnp.float32),
        mesh=mesh,
        scratch_types=[
            pltpu.VMEM((NCH, C), jnp.int32),
            pltpu.VMEM((C,), jnp.float32),
            pltpu.VMEM_SHARED((NP,), jnp.float32),
        ],
    )
    def sc_degree(row_hbm, ones_hbm, zeros_hbm, out_hbm, ridx, ones_v, acc):
        cid = lax.axis_index("c")
        sid = lax.axis_index("s")
        wid = sid * NC + cid
        pltpu.sync_copy(zeros_hbm, acc.at[pl.ds(sid * RPT, RPT)])
        pltpu.sync_copy(row_hbm.at[wid], ridx)
        pltpu.sync_copy(ones_hbm, ones_v)
        plsc.subcore_barrier()

        def body(j, carry):
            pltpu.sync_copy(ones_v, acc.at[ridx.at[j]], add=True)
            return carry

        lax.fori_loop(0, NCH, body, 0)
        plsc.subcore_barrier()
        pltpu.sync_copy(acc.at[pl.ds(sid * RPT, RPT)],
                        out_hbm.at[cid, pl.ds(sid * RPT, RPT)])

    return sc_scatter, sc_degree


def _tc_first(x_ref, aW_ref, ab_ref, W_ref, b_ref, root_ref, degp_ref,
              m_ref, t_ref, dis_ref, deg_ref):
    h = jnp.dot(x_ref[...], aW_ref[...],
                preferred_element_type=jnp.float32) + ab_ref[...]
    h2 = jnp.dot(h, W_ref[...],
                 preferred_element_type=jnp.float32) + b_ref[...]
    deg = degp_ref[0, :N] + degp_ref[1, :N] + 1.0
    dis = lax.rsqrt(deg)
    m_ref[:N, :] = dis[:, None] * jnp.maximum(h2, 0.0)
    m_ref[N:, :] = jnp.zeros((NP - N, D), jnp.float32)
    t_ref[...] = jnp.maximum(h2 + root_ref[...], 0.0) / deg[:, None]
    dis_ref[...] = dis
    deg_ref[...] = deg


def _tc_mid(s_ref, t_ref, dis_ref, deg_ref, gamma_ref, beta_ref,
            W_ref, b_ref, root_ref, m_ref, t2_ref):
    dis = dis_ref[...]
    hp = dis[:, None] * (s_ref[0, :N, :] + s_ref[1, :N, :]) + t_ref[...]
    mean = jnp.mean(hp, axis=0)
    var = jnp.mean((hp - mean) ** 2, axis=0)
    h = gamma_ref[...] * (hp - mean) / jnp.sqrt(var + BN_EPS) + beta_ref[...]
    h = jnp.maximum(h, 0.0)
    h2 = jnp.dot(h, W_ref[...],
                 preferred_element_type=jnp.float32) + b_ref[...]
    m_ref[:N, :] = dis[:, None] * jnp.maximum(h2, 0.0)
    m_ref[N:, :] = jnp.zeros((NP - N, D), jnp.float32)
    t2_ref[...] = jnp.maximum(h2 + root_ref[...], 0.0) / deg_ref[...][:, None]


def _tc_final(s_ref, t_ref, dis_ref, gamma_ref, beta_ref, out_ref):
    dis = dis_ref[...]
    hp = dis[:, None] * (s_ref[0, :N, :] + s_ref[1, :N, :]) + t_ref[...]
    mean = jnp.mean(hp, axis=0)
    var = jnp.mean((hp - mean) ** 2, axis=0)
    out_ref[...] = (gamma_ref[...] * (hp - mean) / jnp.sqrt(var + BN_EPS)
                    + beta_ref[...])


_f32 = jnp.float32
_nd = jax.ShapeDtypeStruct((N, D), _f32)
_npd = jax.ShapeDtypeStruct((NP, D), _f32)
_n1 = jax.ShapeDtypeStruct((N,), _f32)

_tc_first_call = pl.pallas_call(_tc_first, out_shape=[_npd, _nd, _n1, _n1])
_tc_mid_call = pl.pallas_call(_tc_mid, out_shape=[_npd, _nd])
_tc_final_call = pl.pallas_call(_tc_final, out_shape=_nd)


def kernel(x, params, edge_index):
    row = edge_index[0].astype(jnp.int32)
    col = edge_index[1].astype(jnp.int32)
    pad = N + (jnp.arange(EPAD - E, dtype=jnp.int32) % (NP - N))
    row2 = jnp.concatenate([row, pad]).reshape(NW, EPW)
    row3 = row2.reshape(NW, NCH, C)
    col3 = jnp.concatenate([col, pad]).reshape(NW, NCH, C)
    zeros_d = jnp.zeros((RPT, D), _f32)
    zeros_1 = jnp.zeros((RPT,), _f32)
    ones_1 = jnp.ones((C,), _f32)

    sc_scatter, sc_degree = _build_sc_kernels()
    degp = sc_degree(row3, ones_1, zeros_1)
    layers = params['layers']
    m, t, dis, deg = _tc_first_call(
        x, params['atom_W'], params['atom_b'],
        layers[0]['W'], layers[0]['b'], layers[0]['root'], degp)
    for l in range(NUM_LAYERS):
        s = sc_scatter(m, row2, col3, zeros_d)
        lp = layers[l]
        if l < NUM_LAYERS - 1:
            nxt = layers[l + 1]
            m, t = _tc_mid_call(s, t, dis, deg, lp['gamma'], lp['beta'],
                                nxt['W'], nxt['b'], nxt['root'])
        else:
            return _tc_final_call(s, t, dis, lp['gamma'], lp['beta'])


# speedup vs baseline: 1.0053x; 1.0053x over previous
import functools

import jax
import jax.numpy as jnp
from jax import lax
from jax.experimental import pallas as pl
from jax.experimental.pallas import tpu as pltpu
from jax.experimental.pallas import tpu_sc as plsc

N = 10000
D = 128
E = 320000
NUM_LAYERS = 3
BN_EPS = 1e-5

NC = 2
NS = 16
NW = NC * NS
NP = 10240
EPW = 10240
EPAD = NW * EPW
C = 80
NCH = EPW // C
RPT = NP // NS


@functools.cache
def _build_sc_kernels():
    mesh = plsc.VectorSubcoreMesh(core_axis_name="c", subcore_axis_name="s")

    @functools.partial(
        pl.kernel,
        out_type=jax.ShapeDtypeStruct((NC, NP, D), jnp.float32),
        mesh=mesh,
        scratch_types=[
            pltpu.VMEM((EPW,), jnp.int32),
            pltpu.VMEM((NCH, C), jnp.int32),
            pltpu.VMEM((C, D), jnp.float32),
            pltpu.VMEM((C, D), jnp.float32),
            pltpu.VMEM_SHARED((NP, D), jnp.float32),
            pltpu.SemaphoreType.DMA,
            pltpu.SemaphoreType.DMA,
        ],
    )
    def sc_scatter(m_hbm, row_hbm, col_hbm, zeros_hbm, out_hbm,
                   ridx, cidx, g0, g1, acc, sem0, sem1):
        cid = lax.axis_index("c")
        sid = lax.axis_index("s")
        wid = sid * NC + cid
        pltpu.sync_copy(zeros_hbm, acc.at[pl.ds(sid * RPT, RPT)])
        pltpu.sync_copy(row_hbm.at[wid], ridx)
        pltpu.sync_copy(col_hbm.at[wid], cidx)
        plsc.subcore_barrier()

        pltpu.async_copy(m_hbm.at[ridx.at[pl.ds(0, C)]], g0, sem0)

        def body(i, carry):
            ja = 2 * i
            jb = 2 * i + 1
            jn = jnp.minimum(jb + 1, NCH - 1)
            pltpu.make_async_copy(m_hbm.at[ridx.at[pl.ds(ja * C, C)]], g0, sem0).wait()
            pltpu.async_copy(m_hbm.at[ridx.at[pl.ds(jb * C, C)]], g1, sem1)
            pltpu.sync_copy(g0, acc.at[cidx.at[ja]], add=True)
            pltpu.make_async_copy(m_hbm.at[ridx.at[pl.ds(jb * C, C)]], g1, sem1).wait()
            pltpu.async_copy(m_hbm.at[ridx.at[pl.ds(jn * C, C)]], g0, sem0)
            pltpu.sync_copy(g1, acc.at[cidx.at[jb]], add=True)
            return carry

        lax.fori_loop(0, NCH // 2, body, 0)
        pltpu.make_async_copy(m_hbm.at[ridx.at[pl.ds(0, C)]], g0, sem0).wait()
        plsc.subcore_barrier()
        pltpu.sync_copy(acc.at[pl.ds(sid * RPT, RPT)],
                        out_hbm.at[cid, pl.ds(sid * RPT, RPT)])

    @functools.partial(
        pl.kernel,
        out_type=jax.ShapeDtypeStruct((NC, NP), jnp.float32),
        mesh=mesh,
        scratch_types=[
            pltpu.VMEM((NCH, C), jnp.int32),
            pltpu.VMEM((C,), jnp.float32),
            pltpu.VMEM_SHARED((NP,), jnp.float32),
            pltpu.SemaphoreType.DMA,
        ],
    )
    def sc_degree(row_hbm, ones_hbm, zeros_hbm, out_hbm, ridx, ones_v, acc, sem):
        cid = lax.axis_index("c")
        sid = lax.axis_index("s")
        wid = sid * NC + cid
        pltpu.sync_copy(zeros_hbm, acc.at[pl.ds(sid * RPT, RPT)])
        pltpu.sync_copy(row_hbm.at[wid], ridx)
        pltpu.sync_copy(ones_hbm, ones_v)
        plsc.subcore_barrier()

        pltpu.async_copy(ones_v, acc.at[ridx.at[0]], sem, add=True)

        def body(j, carry):
            pltpu.async_copy(ones_v, acc.at[ridx.at[j + 1]], sem, add=True)
            pltpu.make_async_copy(ones_v, acc.at[ridx.at[j]], sem).wait()
            return carry

        lax.fori_loop(0, NCH - 1, body, 0)
        pltpu.make_async_copy(ones_v, acc.at[ridx.at[NCH - 1]], sem).wait()
        plsc.subcore_barrier()
        pltpu.sync_copy(acc.at[pl.ds(sid * RPT, RPT)],
                        out_hbm.at[cid, pl.ds(sid * RPT, RPT)])

    return sc_scatter, sc_degree


def _tc_mm(x_ref, aW_ref, ab_ref, W_ref, b_ref, h2_ref):
    h = jnp.dot(x_ref[...], aW_ref[...],
                preferred_element_type=jnp.float32) + ab_ref[...]
    h2_ref[...] = jnp.dot(h, W_ref[...],
                          preferred_element_type=jnp.float32) + b_ref[...]


def _tc_msg(h2_ref, root_ref, degp_ref, m_ref, t_ref, dis_ref, deg_ref):
    h2 = h2_ref[...]
    deg = degp_ref[0, :N] + degp_ref[1, :N] + 1.0
    dis = lax.rsqrt(deg)
    m_ref[:N, :] = dis[:, None] * jnp.maximum(h2, 0.0)
    m_ref[N:, :] = jnp.zeros((NP - N, D), jnp.float32)
    t_ref[...] = jnp.maximum(h2 + root_ref[...], 0.0) / deg[:, None]
    dis_ref[...] = dis
    deg_ref[...] = deg


def _tc_mid(s_ref, t_ref, dis_ref, deg_ref, gamma_ref, beta_ref,
            W_ref, b_ref, root_ref, m_ref, t2_ref):
    dis = dis_ref[...]
    hp = dis[:, None] * (s_ref[0, :N, :] + s_ref[1, :N, :]) + t_ref[...]
    mean = jnp.mean(hp, axis=0)
    var = jnp.mean((hp - mean) ** 2, axis=0)
    h = gamma_ref[...] * (hp - mean) / jnp.sqrt(var + BN_EPS) + beta_ref[...]
    h = jnp.maximum(h, 0.0)
    h2 = jnp.dot(h, W_ref[...],
                 preferred_element_type=jnp.float32) + b_ref[...]
    m_ref[:N, :] = dis[:, None] * jnp.maximum(h2, 0.0)
    m_ref[N:, :] = jnp.zeros((NP - N, D), jnp.float32)
    t2_ref[...] = jnp.maximum(h2 + root_ref[...], 0.0) / deg_ref[...][:, None]


def _tc_final(s_ref, t_ref, dis_ref, gamma_ref, beta_ref, out_ref):
    dis = dis_ref[...]
    hp = dis[:, None] * (s_ref[0, :N, :] + s_ref[1, :N, :]) + t_ref[...]
    mean = jnp.mean(hp, axis=0)
    var = jnp.mean((hp - mean) ** 2, axis=0)
    out_ref[...] = (gamma_ref[...] * (hp - mean) / jnp.sqrt(var + BN_EPS)
                    + beta_ref[...])


_f32 = jnp.float32
_nd = jax.ShapeDtypeStruct((N, D), _f32)
_npd = jax.ShapeDtypeStruct((NP, D), _f32)
_n1 = jax.ShapeDtypeStruct((N,), _f32)

_tc_mm_call = pl.pallas_call(_tc_mm, out_shape=_nd)
_tc_msg_call = pl.pallas_call(_tc_msg, out_shape=[_npd, _nd, _n1, _n1])
_tc_mid_call = pl.pallas_call(_tc_mid, out_shape=[_npd, _nd])
_tc_final_call = pl.pallas_call(_tc_final, out_shape=_nd)


def kernel(x, params, edge_index):
    row = edge_index[0].astype(jnp.int32)
    col = edge_index[1].astype(jnp.int32)
    pad = N + (jnp.arange(EPAD - E, dtype=jnp.int32) % (NP - N))
    row2 = jnp.concatenate([row, pad]).reshape(NW, EPW)
    row3 = row2.reshape(NW, NCH, C)
    col3 = jnp.concatenate([col, pad]).reshape(NW, NCH, C)
    zeros_d = jnp.zeros((RPT, D), _f32)
    zeros_1 = jnp.zeros((RPT,), _f32)
    ones_1 = jnp.ones((C,), _f32)

    sc_scatter, sc_degree = _build_sc_kernels()
    layers = params['layers']
    degp = sc_degree(row3, ones_1, zeros_1)
    h2 = _tc_mm_call(x, params['atom_W'], params['atom_b'],
                     layers[0]['W'], layers[0]['b'])
    m, t, dis, deg = _tc_msg_call(h2, layers[0]['root'], degp)
    for l in range(NUM_LAYERS):
        s = sc_scatter(m, row2, col3, zeros_d)
        lp = layers[l]
        if l < NUM_LAYERS - 1:
            nxt = layers[l + 1]
            m, t = _tc_mid_call(s, t, dis, deg, lp['gamma'], lp['beta'],
                                nxt['W'], nxt['b'], nxt['root'])
        else:
            return _tc_final_call(s, t, dis, lp['gamma'], lp['beta'])


# speedup vs baseline: 1.0165x; 1.0111x over previous
import functools

import jax
import jax.numpy as jnp
from jax import lax
from jax.experimental import pallas as pl
from jax.experimental.pallas import tpu as pltpu
from jax.experimental.pallas import tpu_sc as plsc

N = 10000
D = 128
E = 320000
NUM_LAYERS = 3
BN_EPS = 1e-5

NC = 2
NS = 16
NW = NC * NS
NP = 10240
EPW = 10240
EPAD = NW * EPW
C = 80
NCH = EPW // C
RPT = NP // NS
DC = 128
DNCH = EPW // DC


@functools.cache
def _build_sc_kernels():
    mesh = plsc.VectorSubcoreMesh(core_axis_name="c", subcore_axis_name="s")

    @functools.partial(
        pl.kernel,
        out_type=jax.ShapeDtypeStruct((NC, NP, D), jnp.float32),
        mesh=mesh,
        scratch_types=[
            pltpu.VMEM((EPW,), jnp.int32),
            pltpu.VMEM((NCH, C), jnp.int32),
            pltpu.VMEM((C, D), jnp.float32),
            pltpu.VMEM((C, D), jnp.float32),
            pltpu.VMEM_SHARED((NP, D), jnp.float32),
            pltpu.SemaphoreType.DMA,
            pltpu.SemaphoreType.DMA,
        ],
    )
    def sc_scatter(m_hbm, row_hbm, col_hbm, zeros_hbm, out_hbm,
                   ridx, cidx, g0, g1, acc, sem0, sem1):
        cid = lax.axis_index("c")
        sid = lax.axis_index("s")
        wid = sid * NC + cid
        pltpu.async_copy(zeros_hbm, acc.at[pl.ds(sid * RPT, RPT)], sem0)
        pltpu.async_copy(row_hbm.at[wid], ridx, sem1)
        pltpu.async_copy(col_hbm.at[wid], cidx, sem1)
        pltpu.make_async_copy(zeros_hbm, acc.at[pl.ds(sid * RPT, RPT)], sem0).wait()
        pltpu.make_async_copy(row_hbm.at[wid], ridx, sem1).wait()
        pltpu.make_async_copy(col_hbm.at[wid], cidx, sem1).wait()
        plsc.subcore_barrier()

        pltpu.async_copy(m_hbm.at[ridx.at[pl.ds(0, C)]], g0, sem0)

        def body(i, carry):
            ja = 2 * i
            jb = 2 * i + 1
            jn = jnp.minimum(jb + 1, NCH - 1)
            pltpu.make_async_copy(m_hbm.at[ridx.at[pl.ds(ja * C, C)]], g0, sem0).wait()
            pltpu.async_copy(m_hbm.at[ridx.at[pl.ds(jb * C, C)]], g1, sem1)
            pltpu.sync_copy(g0, acc.at[cidx.at[ja]], add=True)
            pltpu.make_async_copy(m_hbm.at[ridx.at[pl.ds(jb * C, C)]], g1, sem1).wait()
            pltpu.async_copy(m_hbm.at[ridx.at[pl.ds(jn * C, C)]], g0, sem0)
            pltpu.sync_copy(g1, acc.at[cidx.at[jb]], add=True)
            return carry

        lax.fori_loop(0, NCH // 2, body, 0)
        pltpu.make_async_copy(m_hbm.at[ridx.at[pl.ds(0, C)]], g0, sem0).wait()
        plsc.subcore_barrier()
        pltpu.sync_copy(acc.at[pl.ds(sid * RPT, RPT)],
                        out_hbm.at[cid, pl.ds(sid * RPT, RPT)])

    @functools.partial(
        pl.kernel,
        out_type=jax.ShapeDtypeStruct((NC, NP), jnp.float32),
        mesh=mesh,
        scratch_types=[
            pltpu.VMEM((DNCH, DC), jnp.int32),
            pltpu.VMEM((DC,), jnp.float32),
            pltpu.VMEM_SHARED((NP,), jnp.float32),
            pltpu.SemaphoreType.DMA,
        ],
    )
    def sc_degree(row_hbm, ones_hbm, zeros_hbm, out_hbm, ridx, ones_v, acc, sem):
        cid = lax.axis_index("c")
        sid = lax.axis_index("s")
        wid = sid * NC + cid
        pltpu.sync_copy(zeros_hbm, acc.at[pl.ds(sid * RPT, RPT)])
        pltpu.sync_copy(row_hbm.at[wid], ridx)
        pltpu.sync_copy(ones_hbm, ones_v)
        plsc.subcore_barrier()

        pltpu.async_copy(ones_v, acc.at[ridx.at[0]], sem, add=True)

        def body(j, carry):
            pltpu.async_copy(ones_v, acc.at[ridx.at[j + 1]], sem, add=True)
            pltpu.make_async_copy(ones_v, acc.at[ridx.at[j]], sem).wait()
            return carry

        lax.fori_loop(0, DNCH - 1, body, 0)
        pltpu.make_async_copy(ones_v, acc.at[ridx.at[DNCH - 1]], sem).wait()
        plsc.subcore_barrier()
        pltpu.sync_copy(acc.at[pl.ds(sid * RPT, RPT)],
                        out_hbm.at[cid, pl.ds(sid * RPT, RPT)])

    return sc_scatter, sc_degree


def _tc_mm(x_ref, aW_ref, ab_ref, W_ref, b_ref, h2_ref):
    h = jnp.dot(x_ref[...], aW_ref[...],
                preferred_element_type=jnp.float32) + ab_ref[...]
    h2_ref[...] = jnp.dot(h, W_ref[...],
                          preferred_element_type=jnp.float32) + b_ref[...]


def _tc_msg(h2_ref, root_ref, degp_ref, m_ref, t_ref, dis_ref, deg_ref):
    h2 = h2_ref[...]
    deg = degp_ref[0, :N] + degp_ref[1, :N] + 1.0
    dis = lax.rsqrt(deg)
    m_ref[:N, :] = dis[:, None] * jnp.maximum(h2, 0.0)
    m_ref[N:, :] = jnp.zeros((NP - N, D), jnp.float32)
    t_ref[...] = jnp.maximum(h2 + root_ref[...], 0.0) / deg[:, None]
    dis_ref[...] = dis
    deg_ref[...] = deg


def _tc_mid(s_ref, t_ref, dis_ref, deg_ref, gamma_ref, beta_ref,
            W_ref, b_ref, root_ref, m_ref, t2_ref):
    dis = dis_ref[...]
    hp = dis[:, None] * (s_ref[0, :N, :] + s_ref[1, :N, :]) + t_ref[...]
    mean = jnp.mean(hp, axis=0)
    var = jnp.mean((hp - mean) ** 2, axis=0)
    h = gamma_ref[...] * (hp - mean) / jnp.sqrt(var + BN_EPS) + beta_ref[...]
    h = jnp.maximum(h, 0.0)
    h2 = jnp.dot(h, W_ref[...],
                 preferred_element_type=jnp.float32) + b_ref[...]
    m_ref[:N, :] = dis[:, None] * jnp.maximum(h2, 0.0)
    m_ref[N:, :] = jnp.zeros((NP - N, D), jnp.float32)
    t2_ref[...] = jnp.maximum(h2 + root_ref[...], 0.0) / deg_ref[...][:, None]


def _tc_final(s_ref, t_ref, dis_ref, gamma_ref, beta_ref, out_ref):
    dis = dis_ref[...]
    hp = dis[:, None] * (s_ref[0, :N, :] + s_ref[1, :N, :]) + t_ref[...]
    mean = jnp.mean(hp, axis=0)
    var = jnp.mean((hp - mean) ** 2, axis=0)
    out_ref[...] = (gamma_ref[...] * (hp - mean) / jnp.sqrt(var + BN_EPS)
                    + beta_ref[...])


_f32 = jnp.float32
_nd = jax.ShapeDtypeStruct((N, D), _f32)
_npd = jax.ShapeDtypeStruct((NP, D), _f32)
_n1 = jax.ShapeDtypeStruct((N,), _f32)

_tc_mm_call = pl.pallas_call(_tc_mm, out_shape=_nd)
_tc_msg_call = pl.pallas_call(_tc_msg, out_shape=[_npd, _nd, _n1, _n1])
_tc_mid_call = pl.pallas_call(_tc_mid, out_shape=[_npd, _nd])
_tc_final_call = pl.pallas_call(_tc_final, out_shape=_nd)


def kernel(x, params, edge_index):
    row = edge_index[0].astype(jnp.int32)
    col = edge_index[1].astype(jnp.int32)
    pad = N + (jnp.arange(EPAD - E, dtype=jnp.int32) % (NP - N))
    row2 = jnp.concatenate([row, pad]).reshape(NW, EPW)
    row3 = row2.reshape(NW, DNCH, DC)
    col3 = jnp.concatenate([col, pad]).reshape(NW, NCH, C)
    zeros_d = jnp.zeros((RPT, D), _f32)
    zeros_1 = jnp.zeros((RPT,), _f32)
    ones_1 = jnp.ones((DC,), _f32)

    sc_scatter, sc_degree = _build_sc_kernels()
    layers = params['layers']
    degp = sc_degree(row3, ones_1, zeros_1)
    h2 = _tc_mm_call(x, params['atom_W'], params['atom_b'],
                     layers[0]['W'], layers[0]['b'])
    m, t, dis, deg = _tc_msg_call(h2, layers[0]['root'], degp)
    for l in range(NUM_LAYERS):
        s = sc_scatter(m, row2, col3, zeros_d)
        lp = layers[l]
        if l < NUM_LAYERS - 1:
            nxt = layers[l + 1]
            m, t = _tc_mid_call(s, t, dis, deg, lp['gamma'], lp['beta'],
                                nxt['W'], nxt['b'], nxt['root'])
        else:
            return _tc_final_call(s, t, dis, lp['gamma'], lp['beta'])


# speedup vs baseline: 1.0174x; 1.0010x over previous
import functools

import jax
import jax.numpy as jnp
from jax import lax
from jax.experimental import pallas as pl
from jax.experimental.pallas import tpu as pltpu
from jax.experimental.pallas import tpu_sc as plsc

N = 10000
D = 128
E = 320000
NUM_LAYERS = 3
BN_EPS = 1e-5

NC = 2
NS = 16
NW = NC * NS
NP = 10240
EPW = 10240
EPAD = NW * EPW
C = 80
NCH = EPW // C
RPT = NP // NS
DC = 128
DNCH = EPW // DC


@functools.cache
def _build_sc_kernels():
    mesh = plsc.VectorSubcoreMesh(core_axis_name="c", subcore_axis_name="s")

    @functools.partial(
        pl.kernel,
        out_type=jax.ShapeDtypeStruct((NC, NP, D), jnp.float32),
        mesh=mesh,
        scratch_types=[
            pltpu.VMEM((EPW,), jnp.int32),
            pltpu.VMEM((NCH, C), jnp.int32),
            pltpu.VMEM((C, D), jnp.float32),
            pltpu.VMEM((C, D), jnp.float32),
            pltpu.VMEM_SHARED((NP, D), jnp.float32),
            pltpu.SemaphoreType.DMA,
            pltpu.SemaphoreType.DMA,
        ],
    )
    def sc_scatter(m_hbm, row_hbm, col_hbm, zeros_hbm, out_hbm,
                   ridx, cidx, g0, g1, acc, sem0, sem1):
        cid = lax.axis_index("c")
        sid = lax.axis_index("s")
        wid = sid * NC + cid
        pltpu.async_copy(zeros_hbm, acc.at[pl.ds(sid * RPT, RPT)], sem0)
        pltpu.async_copy(row_hbm.at[wid], ridx, sem1)
        pltpu.async_copy(col_hbm.at[wid], cidx, sem1)
        pltpu.make_async_copy(zeros_hbm, acc.at[pl.ds(sid * RPT, RPT)], sem0).wait()
        pltpu.make_async_copy(row_hbm.at[wid], ridx, sem1).wait()
        pltpu.make_async_copy(col_hbm.at[wid], cidx, sem1).wait()
        plsc.subcore_barrier()

        pltpu.async_copy(m_hbm.at[ridx.at[pl.ds(0, C)]], g0, sem0)

        def body(i, carry):
            ja = 2 * i
            jb = 2 * i + 1
            jn = jnp.minimum(jb + 1, NCH - 1)
            pltpu.make_async_copy(m_hbm.at[ridx.at[pl.ds(ja * C, C)]], g0, sem0).wait()
            pltpu.async_copy(m_hbm.at[ridx.at[pl.ds(jb * C, C)]], g1, sem1)
            pltpu.sync_copy(g0, acc.at[cidx.at[ja]], add=True)
            pltpu.make_async_copy(m_hbm.at[ridx.at[pl.ds(jb * C, C)]], g1, sem1).wait()
            pltpu.async_copy(m_hbm.at[ridx.at[pl.ds(jn * C, C)]], g0, sem0)
            pltpu.sync_copy(g1, acc.at[cidx.at[jb]], add=True)
            return carry

        lax.fori_loop(0, NCH // 2, body, 0)
        pltpu.make_async_copy(m_hbm.at[ridx.at[pl.ds(0, C)]], g0, sem0).wait()
        plsc.subcore_barrier()
        pltpu.sync_copy(acc.at[pl.ds(sid * RPT, RPT)],
                        out_hbm.at[cid, pl.ds(sid * RPT, RPT)])

    @functools.partial(
        pl.kernel,
        out_type=jax.ShapeDtypeStruct((NC, NP), jnp.float32),
        mesh=mesh,
        scratch_types=[
            pltpu.VMEM((DNCH, DC), jnp.int32),
            pltpu.VMEM((DC,), jnp.float32),
            pltpu.VMEM_SHARED((NP,), jnp.float32),
            pltpu.SemaphoreType.DMA,
        ],
    )
    def sc_degree(row_hbm, ones_hbm, zeros_hbm, out_hbm, ridx, ones_v, acc, sem):
        cid = lax.axis_index("c")
        sid = lax.axis_index("s")
        wid = sid * NC + cid
        pltpu.sync_copy(zeros_hbm, acc.at[pl.ds(sid * RPT, RPT)])
        pltpu.sync_copy(row_hbm.at[wid], ridx)
        pltpu.sync_copy(ones_hbm, ones_v)
        plsc.subcore_barrier()

        pltpu.async_copy(ones_v, acc.at[ridx.at[0]], sem, add=True)

        def body(j, carry):
            pltpu.async_copy(ones_v, acc.at[ridx.at[j + 1]], sem, add=True)
            pltpu.make_async_copy(ones_v, acc.at[ridx.at[j]], sem).wait()
            return carry

        lax.fori_loop(0, DNCH - 1, body, 0)
        pltpu.make_async_copy(ones_v, acc.at[ridx.at[DNCH - 1]], sem).wait()
        plsc.subcore_barrier()
        pltpu.sync_copy(acc.at[pl.ds(sid * RPT, RPT)],
                        out_hbm.at[cid, pl.ds(sid * RPT, RPT)])

    return sc_scatter, sc_degree


def _tc_first(x_ref, aW_ref, ab_ref, W_ref, b_ref, root_ref, degp_ref,
              m_ref, t_ref, dis_ref, deg_ref):
    h = jnp.dot(x_ref[...], aW_ref[...],
                preferred_element_type=jnp.float32) + ab_ref[...]
    h2 = jnp.dot(h, W_ref[...],
                 preferred_element_type=jnp.float32) + b_ref[...]
    deg = degp_ref[0, :N] + degp_ref[1, :N] + 1.0
    dis = lax.rsqrt(deg)
    m_ref[:N, :] = dis[:, None] * jnp.maximum(h2, 0.0)
    m_ref[N:, :] = jnp.zeros((NP - N, D), jnp.float32)
    t_ref[...] = jnp.maximum(h2 + root_ref[...], 0.0) / deg[:, None]
    dis_ref[...] = dis
    deg_ref[...] = deg


def _tc_mid(s_ref, t_ref, dis_ref, deg_ref, gamma_ref, beta_ref,
            W_ref, b_ref, root_ref, m_ref, t2_ref):
    dis = dis_ref[...]
    hp = dis[:, None] * (s_ref[0, :N, :] + s_ref[1, :N, :]) + t_ref[...]
    mean = jnp.mean(hp, axis=0)
    var = jnp.mean((hp - mean) ** 2, axis=0)
    h = gamma_ref[...] * (hp - mean) / jnp.sqrt(var + BN_EPS) + beta_ref[...]
    h = jnp.maximum(h, 0.0)
    h2 = jnp.dot(h, W_ref[...],
                 preferred_element_type=jnp.float32) + b_ref[...]
    m_ref[:N, :] = dis[:, None] * jnp.maximum(h2, 0.0)
    m_ref[N:, :] = jnp.zeros((NP - N, D), jnp.float32)
    t2_ref[...] = jnp.maximum(h2 + root_ref[...], 0.0) / deg_ref[...][:, None]


def _tc_final(s_ref, t_ref, dis_ref, gamma_ref, beta_ref, out_ref):
    dis = dis_ref[...]
    hp = dis[:, None] * (s_ref[0, :N, :] + s_ref[1, :N, :]) + t_ref[...]
    mean = jnp.mean(hp, axis=0)
    var = jnp.mean((hp - mean) ** 2, axis=0)
    out_ref[...] = (gamma_ref[...] * (hp - mean) / jnp.sqrt(var + BN_EPS)
                    + beta_ref[...])


_f32 = jnp.float32
_nd = jax.ShapeDtypeStruct((N, D), _f32)
_npd = jax.ShapeDtypeStruct((NP, D), _f32)
_n1 = jax.ShapeDtypeStruct((N,), _f32)

_tc_first_call = pl.pallas_call(_tc_first, out_shape=[_npd, _nd, _n1, _n1])
_tc_mid_call = pl.pallas_call(_tc_mid, out_shape=[_npd, _nd])
_tc_final_call = pl.pallas_call(_tc_final, out_shape=_nd)


def kernel(x, params, edge_index):
    row = edge_index[0].astype(jnp.int32)
    col = edge_index[1].astype(jnp.int32)
    pad = N + (jnp.arange(EPAD - E, dtype=jnp.int32) % (NP - N))
    row2 = jnp.concatenate([row, pad]).reshape(NW, EPW)
    row3 = row2.reshape(NW, DNCH, DC)
    col3 = jnp.concatenate([col, pad]).reshape(NW, NCH, C)
    zeros_d = jnp.zeros((RPT, D), _f32)
    zeros_1 = jnp.zeros((RPT,), _f32)
    ones_1 = jnp.ones((DC,), _f32)

    sc_scatter, sc_degree = _build_sc_kernels()
    layers = params['layers']
    degp = sc_degree(row3, ones_1, zeros_1)
    m, t, dis, deg = _tc_first_call(
        x, params['atom_W'], params['atom_b'],
        layers[0]['W'], layers[0]['b'], layers[0]['root'], degp)
    for l in range(NUM_LAYERS):
        s = sc_scatter(m, row2, col3, zeros_d)
        lp = layers[l]
        if l < NUM_LAYERS - 1:
            nxt = layers[l + 1]
            m, t = _tc_mid_call(s, t, dis, deg, lp['gamma'], lp['beta'],
                                nxt['W'], nxt['b'], nxt['root'])
        else:
            return _tc_final_call(s, t, dis, lp['gamma'], lp['beta'])
